# baseline (speedup 1.0000x reference)
import functools
import math

import jax
import jax.numpy as jnp
from jax import lax
from jax.experimental import pallas as pl
from jax.experimental.pallas import tpu as pltpu
from jax.experimental.pallas import tpu_sc as plsc

_WN = 1000
_TN = 5000
_L = 4
_A = 128
_TP = 5120
_NB = _TP // 128
_RPW = _NB * _L
_LANES = 16
_NTILES = 32
_RPT = 32
_LN3 = math.log(3.0)


def _sc_body(wf_hbm, par_hbm, tau_hbm, out_hbm,
             wf_v, par_v, tau_v, row00, row01, row10, row11, ac_v, ac_s,
             sem_in, sem0, sem1):
    cid = lax.axis_index("c")
    sid = lax.axis_index("s")
    wid = sid * 2 + cid
    base = wid * _RPT

    pltpu.sync_copy(par_hbm, par_v)
    pltpu.sync_copy(tau_hbm, tau_v)
    pltpu.sync_copy(wf_hbm.at[wid], wf_v)

    bvec = par_v[pl.ds(_A, _LANES)]

    for h in range(_RPT // _LANES):
        zvec = bvec
        for ch in range(_A // _LANES):
            pv = par_v[pl.ds(ch * _LANES, _LANES)]
            for j in range(_LANES):
                f = ch * _LANES + j
                zvec = zvec + wf_v[pl.ds(f * _RPT + h * _LANES, _LANES)] * pv[j]
        avec = zvec + _LN3
        cvec = 1.0 / (3.0 * (1.0 + jnp.exp(zvec)))
        ac_v[pl.ds(h * _LANES, _LANES)] = avec
        ac_v[pl.ds(_RPT + h * _LANES, _LANES)] = cvec
        for ln in range(_LANES):
            ac_s[h * _LANES + ln] = avec[ln]
            ac_s[_RPT + h * _LANES + ln] = cvec[ln]

    bufs = ((row00, row01), (row10, row11))
    sems = (sem0, sem1)

    def _dst(i):
        return out_hbm.at[pl.ds(jnp.minimum(i, _WN - 1) * _RPW, _RPW)]

    def _pass(pp, _):
        for slot in range(2):
            bufa, bufb = bufs[slot]
            sem = sems[slot]
            w0 = base + 4 * pp + 2 * slot

            @pl.when(pp >= 1)
            def _wait(bufa=bufa, bufb=bufb, sem=sem, w0=w0):
                pltpu.make_async_copy(bufa, _dst(w0 - 4), sem).wait()
                pltpu.make_async_copy(bufb, _dst(w0 - 3), sem).wait()

            wl = 4 * pp + 2 * slot
            a0 = ac_s[wl]
            c0 = ac_s[_RPT + wl]
            a1 = ac_s[wl + 1]
            c1 = ac_s[_RPT + wl + 1]

            @plsc.parallel_loop(0, _RPW, 1, unroll=4)
            def _row(r, bufa=bufa, bufb=bufb, a0=a0, c0=c0, a1=a1, c1=c1):
                for cj in range(128 // _LANES):
                    sl = pl.ds(cj * _LANES, _LANES)
                    t = tau_v[r, sl]
                    bufa[r, sl] = c0 * jnp.exp(t * a0)
                    bufb[r, sl] = c1 * jnp.exp(t * a1)

            pltpu.async_copy(bufa, _dst(w0), sem)
            pltpu.async_copy(bufb, _dst(w0 + 1), sem)
        return _

    lax.fori_loop(0, _RPT // 4, _pass, None)

    for slot in range(2):
        bufa, bufb = bufs[slot]
        w0 = base + _RPT - 4 + 2 * slot
        pltpu.make_async_copy(bufa, _dst(w0), sems[slot]).wait()
        pltpu.make_async_copy(bufb, _dst(w0 + 1), sems[slot]).wait()


@jax.jit
def _run(wf, par, tau):
    mesh = plsc.VectorSubcoreMesh(core_axis_name="c", subcore_axis_name="s")
    f = functools.partial(
        pl.kernel,
        mesh=mesh,
        out_type=jax.ShapeDtypeStruct((_WN * _RPW, 128), jnp.float32),
        scratch_types=[
            pltpu.VMEM((_A * _RPT,), jnp.float32),
            pltpu.VMEM((_A + _LANES,), jnp.float32),
            pltpu.VMEM((_RPW, 128), jnp.float32),
            pltpu.VMEM((_RPW, 128), jnp.float32),
            pltpu.VMEM((_RPW, 128), jnp.float32),
            pltpu.VMEM((_RPW, 128), jnp.float32),
            pltpu.VMEM((_RPW, 128), jnp.float32),
            pltpu.VMEM((2 * _RPT,), jnp.float32),
            pltpu.SMEM((2 * _RPT,), jnp.float32),
            pltpu.SemaphoreType.DMA,
            pltpu.SemaphoreType.DMA,
            pltpu.SemaphoreType.DMA,
        ],
    )(_sc_body)
    return f(wf, par, tau)


def kernel(inputs, W, b):
    wf = inputs[:_WN, :_A]
    pad = jnp.broadcast_to(wf[_WN - 1], (_NTILES * _RPT - _WN, _A))
    wf = jnp.concatenate([wf, pad])
    wf = wf.reshape(_NTILES, _RPT, _A).transpose(0, 2, 1).reshape(_NTILES, _A * _RPT)
    tau2 = jnp.pad(inputs[_WN:, :_L], ((0, _TP - _TN), (0, 0)))
    tau_p = tau2.reshape(_NB, 128, _L).transpose(0, 2, 1).reshape(_RPW, 128)
    par = jnp.concatenate([W[:, 0], jnp.broadcast_to(b, (_LANES,))])
    out = _run(wf, par, tau_p)
    out = out.reshape(_WN, _NB, _L, 128).transpose(0, 1, 3, 2)
    return out.reshape(_WN, _TP, _L)[:, :_TN, :]
